# baseline (speedup 1.0000x reference)
import jax
import jax.numpy as jnp
from jax.experimental import pallas as pl
from jax.experimental.pallas import tpu as pltpu

K_FRAC = 0.5


def _gather_kernel(idx_ref, subs_ref, repr_ref, out_subs_ref, out_repr_ref):
    out_subs_ref[...] = subs_ref[...]
    out_repr_ref[...] = repr_ref[...]


def kernel(x, subs, W, b):
    n_subs, sub_size = subs.shape
    in_dim = x.shape[1]
    kcount = max(2, int(K_FRAC * n_subs))

    sub_repr = jnp.take(x, subs, axis=0).sum(axis=1)
    weights = (sub_repr @ W.T + b).squeeze(-1)
    scores = jax.nn.sigmoid(weights)
    _, idx = jax.lax.top_k(scores, kcount)

    subs3 = subs.reshape(n_subs, 1, sub_size)
    repr3 = sub_repr.reshape(n_subs, 1, in_dim)
    grid_spec = pltpu.PrefetchScalarGridSpec(
        num_scalar_prefetch=1,
        grid=(kcount,),
        in_specs=[
            pl.BlockSpec((1, 1, sub_size), lambda i, idx_ref: (idx_ref[i], 0, 0)),
            pl.BlockSpec((1, 1, in_dim), lambda i, idx_ref: (idx_ref[i], 0, 0)),
        ],
        out_specs=[
            pl.BlockSpec((1, 1, sub_size), lambda i, idx_ref: (i, 0, 0)),
            pl.BlockSpec((1, 1, in_dim), lambda i, idx_ref: (i, 0, 0)),
        ],
    )
    sel_subs3, sel_repr3 = pl.pallas_call(
        _gather_kernel,
        grid_spec=grid_spec,
        out_shape=[
            jax.ShapeDtypeStruct((kcount, 1, sub_size), subs.dtype),
            jax.ShapeDtypeStruct((kcount, 1, in_dim), sub_repr.dtype),
        ],
    )(idx, subs3, repr3)
    return (sel_subs3.reshape(kcount, sub_size), sel_repr3.reshape(kcount, in_dim))


# speedup vs baseline: 9.8979x; 9.8979x over previous
import dataclasses
import functools

import jax
import jax.numpy as jnp
from jax import lax
from jax.experimental import pallas as pl
from jax.experimental.pallas import tpu as pltpu
from jax.experimental.pallas import tpu_sc as plsc

K_FRAC = 0.5

_NC = 2
_NS = 16
_NW = _NC * _NS
_L = 16
_ROWS_PER_CHUNK = 128


def _sc_compiler_params():
    cp = pltpu.CompilerParams()
    if "needs_layout_passes" in pltpu.CompilerParams.__dataclass_fields__:
        cp = dataclasses.replace(cp, needs_layout_passes=False)
    return cp


def _approx_weights(y, idx_flat, n_subs_pad, sub_size):
    per_w = n_subs_pad // _NW
    groups = per_w // _L
    n_nodes = y.shape[0]
    mesh = plsc.VectorSubcoreMesh(core_axis_name="c", subcore_axis_name="s")

    @functools.partial(
        pl.kernel,
        out_type=jax.ShapeDtypeStruct((n_subs_pad,), jnp.float32),
        mesh=mesh,
        scratch_types=[
            pltpu.VMEM((n_nodes,), jnp.float32),
            pltpu.VMEM((sub_size * per_w,), jnp.int32),
            pltpu.VMEM((per_w,), jnp.float32),
            pltpu.SemaphoreType.DMA,
        ],
        compiler_params=_sc_compiler_params(),
    )
    def k(y_hbm, idx_hbm, out_hbm, y_v, idx_v, acc_v, sem):
        wid = lax.axis_index("s") * _NC + lax.axis_index("c")
        pltpu.async_copy(y_hbm, y_v, sem).wait()
        pltpu.sync_copy(idx_hbm.at[pl.ds(wid * sub_size * per_w,
                                         sub_size * per_w)], idx_v)

        @pl.loop(0, groups)
        def _(g):
            col = g * _L
            acc = plsc.load_gather(y_v, [idx_v[pl.ds(col, _L)]])
            for j in range(1, sub_size):
                acc = acc + plsc.load_gather(
                    y_v, [idx_v[pl.ds(j * per_w + col, _L)]])
            acc_v[pl.ds(col, _L)] = acc

        pltpu.sync_copy(acc_v, out_hbm.at[pl.ds(wid * per_w, per_w)])

    return k(y, idx_flat)


def _row_gather(x, flat_idx):
    n_rows = flat_idx.shape[0]
    in_dim = x.shape[1]
    per_w = n_rows // _NW
    n_chunks = per_w // _ROWS_PER_CHUNK
    mesh = plsc.VectorSubcoreMesh(core_axis_name="c", subcore_axis_name="s")

    @functools.partial(
        pl.kernel,
        out_type=jax.ShapeDtypeStruct((n_rows, in_dim), jnp.float32),
        mesh=mesh,
        scratch_types=[
            pltpu.VMEM((_ROWS_PER_CHUNK,), jnp.int32),
            pltpu.VMEM((_ROWS_PER_CHUNK, in_dim), jnp.float32),
            pltpu.SemaphoreType.DMA,
        ],
    )
    def k(x_hbm, idx_hbm, out_hbm, idx_v, rows_v, sem):
        wid = lax.axis_index("s") * _NC + lax.axis_index("c")
        base = wid * per_w

        @pl.loop(0, n_chunks)
        def _(ch):
            off = base + ch * _ROWS_PER_CHUNK
            pltpu.sync_copy(idx_hbm.at[pl.ds(off, _ROWS_PER_CHUNK)], idx_v)
            pltpu.async_copy(x_hbm.at[idx_v], rows_v, sem).wait()
            pltpu.sync_copy(rows_v, out_hbm.at[pl.ds(off, _ROWS_PER_CHUNK)])

    return k(x, flat_idx)


def kernel(x, subs, W, b):
    n_subs, sub_size = subs.shape
    n_nodes, in_dim = x.shape
    kcount = max(2, int(K_FRAC * n_subs))
    ncand = ((kcount + 128 + 255) // 256) * 256
    subs32 = subs.astype(jnp.int32)

    y = (x @ W.T).reshape(-1)
    n_subs_pad = ((n_subs + _NW * _L - 1) // (_NW * _L)) * (_NW * _L)
    padded = jnp.concatenate(
        [subs32, jnp.zeros((n_subs_pad - n_subs, sub_size), jnp.int32)])
    per_w = n_subs_pad // _NW
    idx_flat = padded.reshape(_NW, per_w, sub_size).transpose(0, 2, 1).reshape(-1)
    wb = _approx_weights(y, idx_flat, n_subs_pad, sub_size)[:n_subs]
    _, cand_idx = lax.top_k(wb, ncand)

    cand_subs = jnp.take(subs32, cand_idx, axis=0)
    rows = _row_gather(x, cand_subs.reshape(-1))
    cand_repr = rows.reshape(ncand, sub_size, in_dim).sum(axis=1)
    cand_w = (cand_repr @ W.T + b).squeeze(-1)
    cand_scores = jax.nn.sigmoid(cand_w)

    full = jnp.full((n_subs,), -1.0, jnp.float32).at[cand_idx].set(cand_scores)
    _, idx = lax.top_k(full, kcount)
    selected_subs = jnp.take(subs, idx, axis=0)
    pos = jnp.zeros((n_subs,), jnp.int32).at[cand_idx].set(
        jnp.arange(ncand, dtype=jnp.int32))
    selected_sub_representations = jnp.take(cand_repr, jnp.take(pos, idx),
                                            axis=0)
    return (selected_subs, selected_sub_representations)


# speedup vs baseline: 10.4260x; 1.0534x over previous
import dataclasses
import functools

import jax
import jax.numpy as jnp
from jax import lax
from jax.experimental import pallas as pl
from jax.experimental.pallas import tpu as pltpu
from jax.experimental.pallas import tpu_sc as plsc

K_FRAC = 0.5

_NC = 2
_NS = 16
_NW = _NC * _NS
_L = 16
_ROWS_PER_CHUNK = 128


def _sc_compiler_params():
    cp = pltpu.CompilerParams()
    if "needs_layout_passes" in pltpu.CompilerParams.__dataclass_fields__:
        cp = dataclasses.replace(cp, needs_layout_passes=False)
    return cp


def _approx_weights(y, idx_flat, n_subs_pad, sub_size):
    per_w = n_subs_pad // _NW
    groups = per_w // _L
    n_nodes = y.shape[0]
    mesh = plsc.VectorSubcoreMesh(core_axis_name="c", subcore_axis_name="s")

    @functools.partial(
        pl.kernel,
        out_type=jax.ShapeDtypeStruct((n_subs_pad,), jnp.float32),
        mesh=mesh,
        scratch_types=[
            pltpu.VMEM((n_nodes,), jnp.float32),
            pltpu.VMEM((sub_size * per_w,), jnp.int32),
            pltpu.VMEM((per_w,), jnp.float32),
            pltpu.SemaphoreType.DMA,
        ],
        compiler_params=_sc_compiler_params(),
    )
    def k(y_hbm, idx_hbm, out_hbm, y_v, idx_v, acc_v, sem):
        wid = lax.axis_index("s") * _NC + lax.axis_index("c")
        pltpu.async_copy(y_hbm, y_v, sem).wait()
        pltpu.sync_copy(idx_hbm.at[pl.ds(wid * sub_size * per_w,
                                         sub_size * per_w)], idx_v)

        @pl.loop(0, groups)
        def _(g):
            col = g * _L
            acc = plsc.load_gather(y_v, [idx_v[pl.ds(col, _L)]])
            for j in range(1, sub_size):
                acc = acc + plsc.load_gather(
                    y_v, [idx_v[pl.ds(j * per_w + col, _L)]])
            acc_v[pl.ds(col, _L)] = acc

        pltpu.sync_copy(acc_v, out_hbm.at[pl.ds(wid * per_w, per_w)])

    return k(y, idx_flat)


def _row_gather(x, flat_idx):
    n_rows = flat_idx.shape[0]
    in_dim = x.shape[1]
    per_w = n_rows // _NW
    n_chunks = per_w // _ROWS_PER_CHUNK
    assert n_chunks % 2 == 0
    mesh = plsc.VectorSubcoreMesh(core_axis_name="c", subcore_axis_name="s")

    @functools.partial(
        pl.kernel,
        out_type=jax.ShapeDtypeStruct((n_rows, in_dim), jnp.float32),
        mesh=mesh,
        scratch_types=[
            pltpu.VMEM((2, _ROWS_PER_CHUNK), jnp.int32),
            pltpu.VMEM((2, _ROWS_PER_CHUNK, in_dim), jnp.float32),
            pltpu.SemaphoreType.DMA,
            pltpu.SemaphoreType.DMA,
        ],
    )
    def k(x_hbm, idx_hbm, out_hbm, idx_v, rows_v, sem0, sem1):
        wid = lax.axis_index("s") * _NC + lax.axis_index("c")
        base = wid * per_w
        sems = (sem0, sem1)

        for b in range(2):
            off = base + b * _ROWS_PER_CHUNK
            pltpu.sync_copy(idx_hbm.at[pl.ds(off, _ROWS_PER_CHUNK)],
                            idx_v.at[b])
            pltpu.async_copy(x_hbm.at[idx_v.at[b]], rows_v.at[b], sems[b])

        @pl.loop(0, n_chunks // 2)
        def _(it):
            ch = it * 2
            for b in range(2):
                pltpu.make_async_copy(x_hbm.at[idx_v.at[b]], rows_v.at[b],
                                      sems[b]).wait()
                off = base + (ch + b) * _ROWS_PER_CHUNK
                pltpu.sync_copy(rows_v.at[b],
                                out_hbm.at[pl.ds(off, _ROWS_PER_CHUNK)])
                nxt = ch + 2 + b

                @pl.when(nxt < n_chunks)
                def _():
                    noff = base + nxt * _ROWS_PER_CHUNK
                    pltpu.sync_copy(idx_hbm.at[pl.ds(noff, _ROWS_PER_CHUNK)],
                                    idx_v.at[b])
                    pltpu.async_copy(x_hbm.at[idx_v.at[b]], rows_v.at[b],
                                     sems[b])

    return k(x, flat_idx)


def kernel(x, subs, W, b):
    n_subs, sub_size = subs.shape
    n_nodes, in_dim = x.shape
    kcount = max(2, int(K_FRAC * n_subs))
    ncand = ((kcount + 128 + 511) // 512) * 512
    subs32 = subs.astype(jnp.int32)

    y = (x @ W.T).reshape(-1)
    n_subs_pad = ((n_subs + _NW * _L - 1) // (_NW * _L)) * (_NW * _L)
    padded = jnp.concatenate(
        [subs32, jnp.zeros((n_subs_pad - n_subs, sub_size), jnp.int32)])
    per_w = n_subs_pad // _NW
    idx_flat = padded.reshape(_NW, per_w, sub_size).transpose(0, 2, 1).reshape(-1)
    wb = _approx_weights(y, idx_flat, n_subs_pad, sub_size)[:n_subs]
    _, cand_idx = lax.top_k(wb, ncand)

    cand_subs = jnp.take(subs32, cand_idx, axis=0)
    rows = _row_gather(x, cand_subs.reshape(-1))
    cand_repr = rows.reshape(ncand, sub_size, in_dim).sum(axis=1)
    cand_w = (cand_repr @ W.T + b).squeeze(-1)
    cand_scores = jax.nn.sigmoid(cand_w)

    full = jnp.full((n_subs,), -1.0, jnp.float32).at[cand_idx].set(cand_scores)
    _, idx = lax.top_k(full, kcount)
    selected_subs = jnp.take(subs, idx, axis=0)
    pos = jnp.zeros((n_subs,), jnp.int32).at[cand_idx].set(
        jnp.arange(ncand, dtype=jnp.int32))
    selected_sub_representations = jnp.take(cand_repr, jnp.take(pos, idx),
                                            axis=0)
    return (selected_subs, selected_sub_representations)


# speedup vs baseline: 10.4721x; 1.0044x over previous
import dataclasses
import functools

import jax
import jax.numpy as jnp
from jax import lax
from jax.experimental import pallas as pl
from jax.experimental.pallas import tpu as pltpu
from jax.experimental.pallas import tpu_sc as plsc

K_FRAC = 0.5

_NC = 2
_NS = 16
_NW = _NC * _NS
_L = 16
_ROWS_PER_CHUNK = 128


def _sc_compiler_params():
    cp = pltpu.CompilerParams()
    if "needs_layout_passes" in pltpu.CompilerParams.__dataclass_fields__:
        cp = dataclasses.replace(cp, needs_layout_passes=False)
    return cp


def _approx_weights(y, idx_flat, n_subs_pad, sub_size):
    per_w = n_subs_pad // _NW
    groups = per_w // _L
    n_nodes = y.shape[0]
    mesh = plsc.VectorSubcoreMesh(core_axis_name="c", subcore_axis_name="s")

    @functools.partial(
        pl.kernel,
        out_type=jax.ShapeDtypeStruct((n_subs_pad,), jnp.float32),
        mesh=mesh,
        scratch_types=[
            pltpu.VMEM((n_nodes,), jnp.float32),
            pltpu.VMEM((sub_size * per_w,), jnp.int32),
            pltpu.VMEM((per_w,), jnp.float32),
            pltpu.SemaphoreType.DMA,
        ],
        compiler_params=_sc_compiler_params(),
    )
    def k(y_hbm, idx_hbm, out_hbm, y_v, idx_v, acc_v, sem):
        wid = lax.axis_index("s") * _NC + lax.axis_index("c")
        pltpu.async_copy(y_hbm, y_v, sem).wait()
        pltpu.sync_copy(idx_hbm.at[pl.ds(wid * sub_size * per_w,
                                         sub_size * per_w)], idx_v)

        @pl.loop(0, groups)
        def _(g):
            col = g * _L
            acc = plsc.load_gather(y_v, [idx_v[pl.ds(col, _L)]])
            for j in range(1, sub_size):
                acc = acc + plsc.load_gather(
                    y_v, [idx_v[pl.ds(j * per_w + col, _L)]])
            acc_v[pl.ds(col, _L)] = acc

        pltpu.sync_copy(acc_v, out_hbm.at[pl.ds(wid * per_w, per_w)])

    return k(y, idx_flat)


def _row_gather(x, flat_idx):
    n_rows = flat_idx.shape[0]
    in_dim = x.shape[1]
    per_w = n_rows // _NW
    n_chunks = per_w // _ROWS_PER_CHUNK
    assert n_chunks % 2 == 0
    mesh = plsc.VectorSubcoreMesh(core_axis_name="c", subcore_axis_name="s")

    @functools.partial(
        pl.kernel,
        out_type=jax.ShapeDtypeStruct((n_rows, in_dim), jnp.float32),
        mesh=mesh,
        scratch_types=[
            pltpu.VMEM((2, _ROWS_PER_CHUNK), jnp.int32),
            pltpu.VMEM((2, _ROWS_PER_CHUNK, in_dim), jnp.float32),
            pltpu.SemaphoreType.DMA,
            pltpu.SemaphoreType.DMA,
            pltpu.SemaphoreType.DMA,
            pltpu.SemaphoreType.DMA,
        ],
    )
    def k(x_hbm, idx_hbm, out_hbm, idx_v, rows_v, gs0, gs1, ws0, ws1):
        wid = lax.axis_index("s") * _NC + lax.axis_index("c")
        base = wid * per_w
        gsems = (gs0, gs1)
        wsems = (ws0, ws1)

        for b in range(2):
            off = base + b * _ROWS_PER_CHUNK
            pltpu.sync_copy(idx_hbm.at[pl.ds(off, _ROWS_PER_CHUNK)],
                            idx_v.at[b])
            pltpu.async_copy(x_hbm.at[idx_v.at[b]], rows_v.at[b], gsems[b])

        @pl.loop(0, n_chunks // 2)
        def _(it):
            ch = it * 2
            for b in range(2):
                off = base + (ch + b) * _ROWS_PER_CHUNK
                pltpu.make_async_copy(x_hbm.at[idx_v.at[b]], rows_v.at[b],
                                      gsems[b]).wait()
                pltpu.async_copy(rows_v.at[b],
                                 out_hbm.at[pl.ds(off, _ROWS_PER_CHUNK)],
                                 wsems[b])
                nxt = ch + 2 + b

                @pl.when(nxt < n_chunks)
                def _():
                    noff = base + nxt * _ROWS_PER_CHUNK
                    pltpu.sync_copy(idx_hbm.at[pl.ds(noff, _ROWS_PER_CHUNK)],
                                    idx_v.at[b])
                    pltpu.make_async_copy(
                        rows_v.at[b],
                        out_hbm.at[pl.ds(base + (ch + b) * _ROWS_PER_CHUNK,
                                         _ROWS_PER_CHUNK)],
                        wsems[b]).wait()
                    pltpu.async_copy(x_hbm.at[idx_v.at[b]], rows_v.at[b],
                                     gsems[b])

                @pl.when(nxt >= n_chunks)
                def _():
                    pltpu.make_async_copy(
                        rows_v.at[b],
                        out_hbm.at[pl.ds(base + (ch + b) * _ROWS_PER_CHUNK,
                                         _ROWS_PER_CHUNK)],
                        wsems[b]).wait()

    return k(x, flat_idx)


def kernel(x, subs, W, b):
    n_subs, sub_size = subs.shape
    n_nodes, in_dim = x.shape
    kcount = max(2, int(K_FRAC * n_subs))
    ncand = ((kcount + 128 + 511) // 512) * 512
    subs32 = subs.astype(jnp.int32)

    y = (x @ W.T).reshape(-1)
    n_subs_pad = ((n_subs + _NW * _L - 1) // (_NW * _L)) * (_NW * _L)
    padded = jnp.concatenate(
        [subs32, jnp.zeros((n_subs_pad - n_subs, sub_size), jnp.int32)])
    per_w = n_subs_pad // _NW
    idx_flat = padded.reshape(_NW, per_w, sub_size).transpose(0, 2, 1).reshape(-1)
    wb = _approx_weights(y, idx_flat, n_subs_pad, sub_size)[:n_subs]
    _, cand_idx = lax.top_k(wb, ncand)

    cand_subs = jnp.take(subs32, cand_idx, axis=0)
    rows = _row_gather(x, cand_subs.reshape(-1))
    cand_repr = rows.reshape(ncand, sub_size, in_dim).sum(axis=1)
    cand_w = (cand_repr @ W.T + b).squeeze(-1)
    cand_scores = jax.nn.sigmoid(cand_w)

    full = jnp.full((n_subs,), -1.0, jnp.float32).at[cand_idx].set(cand_scores)
    _, idx = lax.top_k(full, kcount)
    selected_subs = jnp.take(subs, idx, axis=0)
    pos = jnp.zeros((n_subs,), jnp.int32).at[cand_idx].set(
        jnp.arange(ncand, dtype=jnp.int32))
    selected_sub_representations = jnp.take(cand_repr, jnp.take(pos, idx),
                                            axis=0)
    return (selected_subs, selected_sub_representations)


# speedup vs baseline: 10.4810x; 1.0009x over previous
import dataclasses
import functools

import jax
import jax.numpy as jnp
from jax import lax
from jax.experimental import pallas as pl
from jax.experimental.pallas import tpu as pltpu
from jax.experimental.pallas import tpu_sc as plsc

K_FRAC = 0.5

_NC = 2
_NS = 16
_NW = _NC * _NS
_L = 16
_ROWS_PER_CHUNK = 128


def _sc_compiler_params():
    cp = pltpu.CompilerParams()
    if "needs_layout_passes" in pltpu.CompilerParams.__dataclass_fields__:
        cp = dataclasses.replace(cp, needs_layout_passes=False)
    return cp


def _approx_weights(y, idx_flat, n_subs_pad, sub_size):
    per_w = n_subs_pad // _NW
    groups = per_w // _L
    n_nodes = y.shape[0]
    mesh = plsc.VectorSubcoreMesh(core_axis_name="c", subcore_axis_name="s")

    @functools.partial(
        pl.kernel,
        out_type=jax.ShapeDtypeStruct((n_subs_pad,), jnp.float32),
        mesh=mesh,
        scratch_types=[
            pltpu.VMEM((n_nodes,), jnp.float32),
            pltpu.VMEM((sub_size * per_w,), jnp.int32),
            pltpu.VMEM((per_w,), jnp.float32),
            pltpu.SemaphoreType.DMA,
        ],
        compiler_params=_sc_compiler_params(),
    )
    def k(y_hbm, idx_hbm, out_hbm, y_v, idx_v, acc_v, sem):
        wid = lax.axis_index("s") * _NC + lax.axis_index("c")
        pltpu.async_copy(y_hbm, y_v, sem).wait()
        pltpu.sync_copy(idx_hbm.at[pl.ds(wid * sub_size * per_w,
                                         sub_size * per_w)], idx_v)

        @pl.loop(0, groups)
        def _(g):
            col = g * _L
            acc = plsc.load_gather(y_v, [idx_v[pl.ds(col, _L)]])
            for j in range(1, sub_size):
                acc = acc + plsc.load_gather(
                    y_v, [idx_v[pl.ds(j * per_w + col, _L)]])
            acc_v[pl.ds(col, _L)] = acc

        pltpu.sync_copy(acc_v, out_hbm.at[pl.ds(wid * per_w, per_w)])

    return k(y, idx_flat)


_NB = 4
_CHUNK = 64


def _row_gather(x, flat_idx):
    n_rows = flat_idx.shape[0]
    in_dim = x.shape[1]
    per_w = n_rows // _NW
    n_chunks = per_w // _CHUNK
    assert n_chunks % _NB == 0
    mesh = plsc.VectorSubcoreMesh(core_axis_name="c", subcore_axis_name="s")

    @functools.partial(
        pl.kernel,
        out_type=jax.ShapeDtypeStruct((n_rows, in_dim), jnp.float32),
        mesh=mesh,
        scratch_types=[
            pltpu.VMEM((_NB, _CHUNK), jnp.int32),
            pltpu.VMEM((_NB, _CHUNK, in_dim), jnp.float32),
        ] + [pltpu.SemaphoreType.DMA] * (2 * _NB),
    )
    def k(x_hbm, idx_hbm, out_hbm, idx_v, rows_v, *sems):
        gsems, wsems = sems[:_NB], sems[_NB:]
        wid = lax.axis_index("s") * _NC + lax.axis_index("c")
        base = wid * per_w

        for b in range(_NB):
            off = base + b * _CHUNK
            pltpu.sync_copy(idx_hbm.at[pl.ds(off, _CHUNK)], idx_v.at[b])
            pltpu.async_copy(x_hbm.at[idx_v.at[b]], rows_v.at[b], gsems[b])

        @pl.loop(0, n_chunks // _NB)
        def _(it):
            ch = it * _NB
            for b in range(_NB):
                off = base + (ch + b) * _CHUNK
                pltpu.make_async_copy(x_hbm.at[idx_v.at[b]], rows_v.at[b],
                                      gsems[b]).wait()
                pltpu.async_copy(rows_v.at[b], out_hbm.at[pl.ds(off, _CHUNK)],
                                 wsems[b])
                nxt = ch + _NB + b

                @pl.when(nxt < n_chunks)
                def _():
                    noff = base + nxt * _CHUNK
                    pltpu.sync_copy(idx_hbm.at[pl.ds(noff, _CHUNK)],
                                    idx_v.at[b])
                    pltpu.make_async_copy(rows_v.at[b],
                                          out_hbm.at[pl.ds(off, _CHUNK)],
                                          wsems[b]).wait()
                    pltpu.async_copy(x_hbm.at[idx_v.at[b]], rows_v.at[b],
                                     gsems[b])

                @pl.when(nxt >= n_chunks)
                def _():
                    pltpu.make_async_copy(rows_v.at[b],
                                          out_hbm.at[pl.ds(off, _CHUNK)],
                                          wsems[b]).wait()

    return k(x, flat_idx)


def kernel(x, subs, W, b):
    n_subs, sub_size = subs.shape
    n_nodes, in_dim = x.shape
    kcount = max(2, int(K_FRAC * n_subs))
    ncand = ((kcount + 128 + 511) // 512) * 512
    subs32 = subs.astype(jnp.int32)

    y = (x @ W.T).reshape(-1)
    n_subs_pad = ((n_subs + _NW * _L - 1) // (_NW * _L)) * (_NW * _L)
    padded = jnp.concatenate(
        [subs32, jnp.zeros((n_subs_pad - n_subs, sub_size), jnp.int32)])
    per_w = n_subs_pad // _NW
    idx_flat = padded.reshape(_NW, per_w, sub_size).transpose(0, 2, 1).reshape(-1)
    wb = _approx_weights(y, idx_flat, n_subs_pad, sub_size)[:n_subs]
    _, cand_idx = lax.top_k(wb, ncand)

    cand_subs = jnp.take(subs32, cand_idx, axis=0)
    rows = _row_gather(x, cand_subs.reshape(-1))
    cand_repr = rows.reshape(ncand, sub_size, in_dim).sum(axis=1)
    cand_w = (cand_repr @ W.T + b).squeeze(-1)
    cand_scores = jax.nn.sigmoid(cand_w)

    full = jnp.full((n_subs,), -1.0, jnp.float32).at[cand_idx].set(cand_scores)
    _, idx = lax.top_k(full, kcount)
    selected_subs = jnp.take(subs, idx, axis=0)
    pos = jnp.zeros((n_subs,), jnp.int32).at[cand_idx].set(
        jnp.arange(ncand, dtype=jnp.int32))
    selected_sub_representations = jnp.take(cand_repr, jnp.take(pos, idx),
                                            axis=0)
    return (selected_subs, selected_sub_representations)
